# Initial kernel scaffold; baseline (speedup 1.0000x reference)
import jax
import jax.numpy as jnp
from jax.experimental import pallas as pl


def kernel(ft_a, ft_p, edge_a2p, edge_p2a, W0_ap, W0_pa, W1_ap, W1_pa, Wfc_a, bfc_a, Wfc_p, bfc_p):
    raise NotImplementedError("write your pallas kernel here")


# speedup vs baseline: 2.4144x; 2.4144x over previous
import jax
import jax.numpy as jnp
from jax import lax
from jax.experimental import pallas as pl
from jax.experimental.pallas import tpu as pltpu
from jax.experimental.pallas import tpu_sc as plsc

N_NODES = 10000
FEAT = 128
ACC_ROWS = 10240
ROWS_PER_TILE = ACC_ROWS // 16
PAD_DST = 10200
CHUNK = 128
N_CHUNKS = 160
E_PER_TILE = CHUNK * N_CHUNKS
E_PAD = 16 * E_PER_TILE

_MESH = plsc.VectorSubcoreMesh(core_axis_name="c", subcore_axis_name="s")


def _sc_agg():
    out_type = [jax.ShapeDtypeStruct((2 * ACC_ROWS, FEAT), jnp.float32)]
    scratch = [
        pltpu.VMEM_SHARED((ACC_ROWS, FEAT), jnp.float32),
        pltpu.VMEM((CHUNK,), jnp.int32),
        pltpu.VMEM((CHUNK,), jnp.int32),
        pltpu.VMEM((CHUNK, FEAT), jnp.float32),
    ]

    def body(table, src_hbm, dst_hbm, zfeat, sums_out, acc_sh, src_v, dst_v,
             rows_v):
        c = lax.axis_index("c")
        s = lax.axis_index("s")
        tid = c * 16 + s
        row0 = s * ROWS_PER_TILE
        nsub = ROWS_PER_TILE // CHUNK
        pltpu.sync_copy(zfeat, rows_v)

        def zblk(k, carry):
            pltpu.sync_copy(rows_v, acc_sh.at[pl.ds(row0 + k * CHUNK, CHUNK)])
            return carry

        lax.fori_loop(0, nsub, zblk, 0)
        plsc.subcore_barrier()

        def chunk(j, carry):
            pltpu.sync_copy(src_hbm.at[tid * N_CHUNKS + j], src_v)
            pltpu.sync_copy(dst_hbm.at[tid * N_CHUNKS + j], dst_v)
            pltpu.sync_copy(table.at[src_v], rows_v)
            pltpu.sync_copy(rows_v, acc_sh.at[dst_v], add=True)
            return carry

        lax.fori_loop(0, N_CHUNKS, chunk, 0)
        plsc.subcore_barrier()
        out0 = c * ACC_ROWS + row0

        def oblk(k, carry):
            pltpu.sync_copy(acc_sh.at[pl.ds(row0 + k * CHUNK, CHUNK)], rows_v)
            pltpu.sync_copy(rows_v, sums_out.at[pl.ds(out0 + k * CHUNK, CHUNK)])
            return carry

        lax.fori_loop(0, nsub, oblk, 0)

    return pl.kernel(body, mesh=_MESH, out_type=out_type, scratch_types=scratch)


def _sc_cnt():
    out_type = [jax.ShapeDtypeStruct((2 * ACC_ROWS, FEAT), jnp.float32)]
    scratch = [
        pltpu.VMEM_SHARED((ACC_ROWS, FEAT), jnp.float32),
        pltpu.VMEM((CHUNK,), jnp.int32),
        pltpu.VMEM((CHUNK, FEAT), jnp.float32),
    ]

    def body(dst_hbm, zfeat, ones_hbm, cnts_out, acc_sh, dst_v, rows_v):
        c = lax.axis_index("c")
        s = lax.axis_index("s")
        tid = c * 16 + s
        row0 = s * ROWS_PER_TILE
        nsub = ROWS_PER_TILE // CHUNK
        pltpu.sync_copy(zfeat, rows_v)

        def zblk(k, carry):
            pltpu.sync_copy(rows_v, acc_sh.at[pl.ds(row0 + k * CHUNK, CHUNK)])
            return carry

        lax.fori_loop(0, nsub, zblk, 0)
        pltpu.sync_copy(ones_hbm, rows_v)
        plsc.subcore_barrier()

        def chunk(j, carry):
            pltpu.sync_copy(dst_hbm.at[tid * N_CHUNKS + j], dst_v)
            pltpu.sync_copy(rows_v, acc_sh.at[dst_v], add=True)
            return carry

        lax.fori_loop(0, N_CHUNKS, chunk, 0)
        plsc.subcore_barrier()
        out0 = c * ACC_ROWS + row0

        def oblk(k, carry):
            pltpu.sync_copy(acc_sh.at[pl.ds(row0 + k * CHUNK, CHUNK)], rows_v)
            pltpu.sync_copy(rows_v, cnts_out.at[pl.ds(out0 + k * CHUNK, CHUNK)])
            return carry

        lax.fori_loop(0, nsub, oblk, 0)

    return pl.kernel(body, mesh=_MESH, out_type=out_type, scratch_types=scratch)


def _tc1_body(sums_ref, cnts_ref, w_ref, out_ref):
    s = sums_ref[0][:N_NODES]
    cnt = cnts_ref[0][:N_NODES, 0:1]
    m = s / jnp.maximum(cnt, 1.0)
    out_ref[0] = jnp.maximum(
        jnp.dot(m, w_ref[0], preferred_element_type=jnp.float32), 0.0)


def _tc2_body(sums_ref, cnts_ref, w1_ref, wfc_ref, b_ref, ft_ref, out_ref):
    s = sums_ref[0][:N_NODES]
    cnt = cnts_ref[0][:N_NODES, 0:1]
    m = s / jnp.maximum(cnt, 1.0)
    v = jnp.maximum(jnp.dot(m, w1_ref[0], preferred_element_type=jnp.float32), 0.0)
    o = (jnp.dot(v, wfc_ref[0][:FEAT], preferred_element_type=jnp.float32)
         + jnp.dot(ft_ref[0], wfc_ref[0][FEAT:], preferred_element_type=jnp.float32)
         + b_ref[0, 0])
    out_ref[0] = o


def _pad_edges(idx, pad_val):
    pad = jnp.full((E_PAD - idx.shape[0],), pad_val, jnp.int32)
    return jnp.concatenate([idx.astype(jnp.int32), pad]).reshape(
        16 * N_CHUNKS, CHUNK)


def kernel(ft_a, ft_p, edge_a2p, edge_p2a, W0_ap, W0_pa, W1_ap, W1_pa,
           Wfc_a, bfc_a, Wfc_p, bfc_p):
    f32 = jnp.float32
    src_all = jnp.concatenate([
        _pad_edges(edge_a2p[1], 0),
        _pad_edges(edge_p2a[1] + N_NODES, N_NODES),
    ])
    dst_all = jnp.concatenate([
        _pad_edges(edge_a2p[0], PAD_DST),
        _pad_edges(edge_p2a[0], PAD_DST),
    ])

    zfeat = jnp.zeros((CHUNK, FEAT), f32)
    ones = jnp.ones((CHUNK, FEAT), f32)

    table1 = jnp.concatenate([ft_p, ft_a], axis=0)
    sums1 = _sc_agg()(table1, src_all, dst_all, zfeat)[0]
    cnts = _sc_cnt()(dst_all, zfeat, ones)[0]

    sums1_r = sums1.reshape(2, ACC_ROWS, FEAT)
    cnts_r = cnts.reshape(2, ACC_ROWS, FEAT)
    w0 = jnp.stack([W0_pa, W0_ap]).reshape(2, FEAT, FEAT)
    flip = lambda i: (1 - i, 0, 0)
    ident = lambda i: (i, 0, 0)
    table2 = pl.pallas_call(
        _tc1_body,
        grid=(2,),
        in_specs=[
            pl.BlockSpec((1, ACC_ROWS, FEAT), flip),
            pl.BlockSpec((1, ACC_ROWS, FEAT), flip),
            pl.BlockSpec((1, FEAT, FEAT), ident),
        ],
        out_specs=pl.BlockSpec((1, N_NODES, FEAT), ident),
        out_shape=jax.ShapeDtypeStruct((2, N_NODES, FEAT), f32),
    )(sums1_r, cnts_r, w0)

    sums2 = _sc_agg()(table2.reshape(2 * N_NODES, FEAT), src_all, dst_all,
                      zfeat)[0]

    sums2_r = sums2.reshape(2, ACC_ROWS, FEAT)
    w1 = jnp.stack([W1_ap, W1_pa])
    wfc = jnp.stack([Wfc_a, Wfc_p])
    bfc = jnp.stack([bfc_a, bfc_p]).reshape(2, 1, FEAT)
    ft = jnp.stack([ft_a, ft_p])
    out = pl.pallas_call(
        _tc2_body,
        grid=(2,),
        in_specs=[
            pl.BlockSpec((1, ACC_ROWS, FEAT), ident),
            pl.BlockSpec((1, ACC_ROWS, FEAT), ident),
            pl.BlockSpec((1, FEAT, FEAT), ident),
            pl.BlockSpec((1, 2 * FEAT, FEAT), ident),
            pl.BlockSpec((1, 1, FEAT), ident),
            pl.BlockSpec((1, N_NODES, FEAT), ident),
        ],
        out_specs=pl.BlockSpec((1, N_NODES, FEAT), ident),
        out_shape=jax.ShapeDtypeStruct((2, N_NODES, FEAT), f32),
    )(sums2_r, cnts_r, w1, wfc, bfc, ft)
    return out.reshape(2 * N_NODES, FEAT)


# speedup vs baseline: 3.1357x; 1.2988x over previous
import jax
import jax.numpy as jnp
from jax import lax
from jax.experimental import pallas as pl
from jax.experimental.pallas import tpu as pltpu
from jax.experimental.pallas import tpu_sc as plsc

N_NODES = 10000
FEAT = 128
ACC_ROWS = 10240
ROWS_PER_TILE = ACC_ROWS // 16
PAD_DST = 10200
CHUNK = 128
N_CHUNKS = 160
E_PER_TILE = CHUNK * N_CHUNKS
E_PAD = 16 * E_PER_TILE

_MESH = plsc.VectorSubcoreMesh(core_axis_name="c", subcore_axis_name="s")


def _sc_agg():
    out_type = [jax.ShapeDtypeStruct((2 * ACC_ROWS, FEAT), jnp.float32)]
    scratch = [
        pltpu.VMEM_SHARED((ACC_ROWS, FEAT), jnp.float32),
        pltpu.VMEM((CHUNK,), jnp.int32),
        pltpu.VMEM((CHUNK,), jnp.int32),
        pltpu.VMEM((CHUNK,), jnp.int32),
        pltpu.VMEM((CHUNK,), jnp.int32),
        pltpu.VMEM((CHUNK, FEAT), jnp.float32),
        pltpu.VMEM((CHUNK, FEAT), jnp.float32),
    ] + [pltpu.SemaphoreType.DMA] * 8

    def body(table, src_hbm, dst_hbm, zfeat, sums_out, acc_sh, sv0, sv1, dv0,
             dv1, rw0, rw1, qs0, qs1, qd0, qd1, qg0, qg1, qc0, qc1):
        srcv, dstv, rows = (sv0, sv1), (dv0, dv1), (rw0, rw1)
        si_s, si_d, sg, ss = (qs0, qs1), (qd0, qd1), (qg0, qg1), (qc0, qc1)
        c = lax.axis_index("c")
        s = lax.axis_index("s")
        tid = c * 16 + s
        row0 = s * ROWS_PER_TILE
        nsub = ROWS_PER_TILE // CHUNK

        def issue_src(m, p):
            pltpu.async_copy(src_hbm.at[tid * N_CHUNKS + m], srcv[p], si_s[p])

        def wait_src(p):
            pltpu.make_async_copy(src_hbm.at[0], srcv[p], si_s[p]).wait()

        def issue_dst(m, p):
            pltpu.async_copy(dst_hbm.at[tid * N_CHUNKS + m], dstv[p], si_d[p])

        def wait_dst(p):
            pltpu.make_async_copy(dst_hbm.at[0], dstv[p], si_d[p]).wait()

        def issue_gather(p):
            pltpu.async_copy(table.at[srcv[p]], rows[p], sg[p])

        def wait_gather(p):
            pltpu.make_async_copy(table.at[srcv[p]], rows[p], sg[p]).wait()

        def issue_scatter(p):
            pltpu.async_copy(rows[p], acc_sh.at[dstv[p]], ss[p], add=True)

        def wait_scatter(p):
            pltpu.make_async_copy(rows[p], acc_sh.at[dstv[p]], ss[p]).wait()

        pltpu.sync_copy(zfeat, rw0)

        def zblk(k, carry):
            pltpu.sync_copy(rw0, acc_sh.at[pl.ds(row0 + k * CHUNK, CHUNK)])
            return carry

        lax.fori_loop(0, nsub, zblk, 0)
        plsc.subcore_barrier()

        issue_src(0, 0)
        issue_dst(0, 0)
        issue_src(1, 1)
        wait_src(0)
        issue_gather(0)
        wait_gather(0)
        issue_src(2, 0)
        wait_dst(0)
        issue_scatter(0)
        wait_src(1)
        issue_dst(1, 1)
        issue_gather(1)

        def pair(mm, carry):
            for t in range(2):
                p, q = 1 - t, t
                m = 2 * mm + 1 + t
                wait_gather(p)
                issue_src(m + 2, p)
                wait_dst(p)
                issue_scatter(p)
                wait_src(q)
                wait_scatter(q)
                issue_dst(m + 1, q)
                issue_gather(q)
            return carry

        lax.fori_loop(0, (N_CHUNKS - 2) // 2, pair, 0)
        wait_gather(1)
        wait_dst(1)
        issue_scatter(1)
        wait_scatter(0)
        wait_scatter(1)
        wait_src(0)
        plsc.subcore_barrier()
        out0 = c * ACC_ROWS + row0

        def oblk(k, carry):
            pltpu.sync_copy(acc_sh.at[pl.ds(row0 + k * CHUNK, CHUNK)], rw0)
            pltpu.sync_copy(rw0, sums_out.at[pl.ds(out0 + k * CHUNK, CHUNK)])
            return carry

        lax.fori_loop(0, nsub, oblk, 0)

    return pl.kernel(body, mesh=_MESH, out_type=out_type, scratch_types=scratch)


def _sc_cnt():
    out_type = [jax.ShapeDtypeStruct((2 * ACC_ROWS, FEAT), jnp.float32)]
    scratch = [
        pltpu.VMEM_SHARED((ACC_ROWS, FEAT), jnp.float32),
        pltpu.VMEM((CHUNK,), jnp.int32),
        pltpu.VMEM((CHUNK,), jnp.int32),
        pltpu.VMEM((CHUNK, FEAT), jnp.float32),
    ] + [pltpu.SemaphoreType.DMA] * 4

    def body(dst_hbm, zfeat, ones_hbm, cnts_out, acc_sh, dv0, dv1, rows_v,
             qd0, qd1, qc0, qc1):
        dstv, si_d, ss = (dv0, dv1), (qd0, qd1), (qc0, qc1)
        c = lax.axis_index("c")
        s = lax.axis_index("s")
        tid = c * 16 + s
        row0 = s * ROWS_PER_TILE
        nsub = ROWS_PER_TILE // CHUNK

        def issue_dst(m, p):
            pltpu.async_copy(dst_hbm.at[tid * N_CHUNKS + m], dstv[p], si_d[p])

        def wait_dst(p):
            pltpu.make_async_copy(dst_hbm.at[0], dstv[p], si_d[p]).wait()

        def issue_scatter(p):
            pltpu.async_copy(rows_v, acc_sh.at[dstv[p]], ss[p], add=True)

        def wait_scatter(p):
            pltpu.make_async_copy(rows_v, acc_sh.at[dstv[p]], ss[p]).wait()

        pltpu.sync_copy(zfeat, rows_v)

        def zblk(k, carry):
            pltpu.sync_copy(rows_v, acc_sh.at[pl.ds(row0 + k * CHUNK, CHUNK)])
            return carry

        lax.fori_loop(0, nsub, zblk, 0)
        pltpu.sync_copy(ones_hbm, rows_v)
        plsc.subcore_barrier()

        issue_dst(0, 0)
        wait_dst(0)
        issue_scatter(0)
        issue_dst(1, 1)

        def pair(mm, carry):
            for t in range(2):
                p, q = 1 - t, t
                wait_dst(p)
                issue_scatter(p)
                wait_scatter(q)
                issue_dst(2 * mm + 2 + t, q)
            return carry

        lax.fori_loop(0, (N_CHUNKS - 2) // 2, pair, 0)
        wait_dst(1)
        issue_scatter(1)
        wait_scatter(0)
        wait_scatter(1)
        plsc.subcore_barrier()
        out0 = c * ACC_ROWS + row0

        def oblk(k, carry):
            pltpu.sync_copy(acc_sh.at[pl.ds(row0 + k * CHUNK, CHUNK)], rows_v)
            pltpu.sync_copy(rows_v, cnts_out.at[pl.ds(out0 + k * CHUNK, CHUNK)])
            return carry

        lax.fori_loop(0, nsub, oblk, 0)

    return pl.kernel(body, mesh=_MESH, out_type=out_type, scratch_types=scratch)


def _tc1_body(sums_ref, cnts_ref, w_ref, out_ref):
    s = sums_ref[0][:N_NODES]
    cnt = cnts_ref[0][:N_NODES, 0:1]
    m = s / jnp.maximum(cnt, 1.0)
    out_ref[0] = jnp.maximum(
        jnp.dot(m, w_ref[0], preferred_element_type=jnp.float32), 0.0)


def _tc2_body(sums_ref, cnts_ref, w1_ref, wfc_ref, b_ref, ft_ref, out_ref):
    s = sums_ref[0][:N_NODES]
    cnt = cnts_ref[0][:N_NODES, 0:1]
    m = s / jnp.maximum(cnt, 1.0)
    v = jnp.maximum(jnp.dot(m, w1_ref[0], preferred_element_type=jnp.float32), 0.0)
    o = (jnp.dot(v, wfc_ref[0][:FEAT], preferred_element_type=jnp.float32)
         + jnp.dot(ft_ref[0], wfc_ref[0][FEAT:], preferred_element_type=jnp.float32)
         + b_ref[0, 0])
    out_ref[0] = o


def _pad_edges(idx, pad_val):
    pad = jnp.full((E_PAD - idx.shape[0],), pad_val, jnp.int32)
    return jnp.concatenate([idx.astype(jnp.int32), pad]).reshape(
        16 * N_CHUNKS, CHUNK)


def kernel(ft_a, ft_p, edge_a2p, edge_p2a, W0_ap, W0_pa, W1_ap, W1_pa,
           Wfc_a, bfc_a, Wfc_p, bfc_p):
    f32 = jnp.float32
    src_all = jnp.concatenate([
        _pad_edges(edge_a2p[1], 0),
        _pad_edges(edge_p2a[1] + N_NODES, N_NODES),
        jnp.zeros((16, CHUNK), jnp.int32),
    ])
    dst_all = jnp.concatenate([
        _pad_edges(edge_a2p[0], PAD_DST),
        _pad_edges(edge_p2a[0], PAD_DST),
    ])

    zfeat = jnp.zeros((CHUNK, FEAT), f32)
    ones = jnp.ones((CHUNK, FEAT), f32)

    table1 = jnp.concatenate([ft_p, ft_a], axis=0)
    sums1 = _sc_agg()(table1, src_all, dst_all, zfeat)[0]
    cnts = _sc_cnt()(dst_all, zfeat, ones)[0]

    sums1_r = sums1.reshape(2, ACC_ROWS, FEAT)
    cnts_r = cnts.reshape(2, ACC_ROWS, FEAT)
    w0 = jnp.stack([W0_pa, W0_ap]).reshape(2, FEAT, FEAT)
    flip = lambda i: (1 - i, 0, 0)
    ident = lambda i: (i, 0, 0)
    table2 = pl.pallas_call(
        _tc1_body,
        grid=(2,),
        in_specs=[
            pl.BlockSpec((1, ACC_ROWS, FEAT), flip),
            pl.BlockSpec((1, ACC_ROWS, FEAT), flip),
            pl.BlockSpec((1, FEAT, FEAT), ident),
        ],
        out_specs=pl.BlockSpec((1, N_NODES, FEAT), ident),
        out_shape=jax.ShapeDtypeStruct((2, N_NODES, FEAT), f32),
    )(sums1_r, cnts_r, w0)

    sums2 = _sc_agg()(table2.reshape(2 * N_NODES, FEAT), src_all, dst_all,
                      zfeat)[0]

    sums2_r = sums2.reshape(2, ACC_ROWS, FEAT)
    w1 = jnp.stack([W1_ap, W1_pa])
    wfc = jnp.stack([Wfc_a, Wfc_p])
    bfc = jnp.stack([bfc_a, bfc_p]).reshape(2, 1, FEAT)
    ft = jnp.stack([ft_a, ft_p])
    out = pl.pallas_call(
        _tc2_body,
        grid=(2,),
        in_specs=[
            pl.BlockSpec((1, ACC_ROWS, FEAT), ident),
            pl.BlockSpec((1, ACC_ROWS, FEAT), ident),
            pl.BlockSpec((1, FEAT, FEAT), ident),
            pl.BlockSpec((1, 2 * FEAT, FEAT), ident),
            pl.BlockSpec((1, 1, FEAT), ident),
            pl.BlockSpec((1, N_NODES, FEAT), ident),
        ],
        out_specs=pl.BlockSpec((1, N_NODES, FEAT), ident),
        out_shape=jax.ShapeDtypeStruct((2, N_NODES, FEAT), f32),
    )(sums2_r, cnts_r, w1, wfc, bfc, ft)
    return out.reshape(2 * N_NODES, FEAT)


# speedup vs baseline: 3.4664x; 1.1054x over previous
import jax
import jax.numpy as jnp
from jax import lax
from jax.experimental import pallas as pl
from jax.experimental.pallas import tpu as pltpu
from jax.experimental.pallas import tpu_sc as plsc

N_NODES = 10000
FEAT = 128
ACC_ROWS = 10240
ROWS_PER_TILE = ACC_ROWS // 16
PAD_DST = 10200
CHUNK = 128
N_CHUNKS = 160
E_PER_TILE = CHUNK * N_CHUNKS
E_PAD = 16 * E_PER_TILE

_MESH = plsc.VectorSubcoreMesh(core_axis_name="c", subcore_axis_name="s")


def _sc_agg():
    out_type = [jax.ShapeDtypeStruct((2 * ACC_ROWS, FEAT), jnp.float32)]
    scratch = [
        pltpu.VMEM_SHARED((ACC_ROWS, FEAT), jnp.float32),
        pltpu.VMEM((CHUNK,), jnp.int32),
        pltpu.VMEM((CHUNK,), jnp.int32),
        pltpu.VMEM((CHUNK,), jnp.int32),
        pltpu.VMEM((CHUNK,), jnp.int32),
        pltpu.VMEM((CHUNK, FEAT), jnp.float32),
        pltpu.VMEM((CHUNK, FEAT), jnp.float32),
    ] + [pltpu.SemaphoreType.DMA] * 8

    def body(table, src_hbm, dst_hbm, zfeat, sums_out, acc_sh, sv0, sv1, dv0,
             dv1, rw0, rw1, qs0, qs1, qd0, qd1, qg0, qg1, qc0, qc1):
        srcv, dstv, rows = (sv0, sv1), (dv0, dv1), (rw0, rw1)
        si_s, si_d, sg, ss = (qs0, qs1), (qd0, qd1), (qg0, qg1), (qc0, qc1)
        c = lax.axis_index("c")
        s = lax.axis_index("s")
        tid = c * 16 + s
        row0 = s * ROWS_PER_TILE
        nsub = ROWS_PER_TILE // CHUNK

        def issue_src(m, p):
            pltpu.async_copy(src_hbm.at[tid * N_CHUNKS + m], srcv[p], si_s[p])

        def wait_src(p):
            pltpu.make_async_copy(src_hbm.at[0], srcv[p], si_s[p]).wait()

        def issue_dst(m, p):
            pltpu.async_copy(dst_hbm.at[tid * N_CHUNKS + m], dstv[p], si_d[p])

        def wait_dst(p):
            pltpu.make_async_copy(dst_hbm.at[0], dstv[p], si_d[p]).wait()

        def issue_gather(p):
            pltpu.async_copy(table.at[srcv[p]], rows[p], sg[p])

        def wait_gather(p):
            pltpu.make_async_copy(table.at[srcv[p]], rows[p], sg[p]).wait()

        def issue_scatter(p):
            pltpu.async_copy(rows[p], acc_sh.at[dstv[p]], ss[p], add=True)

        def wait_scatter(p):
            pltpu.make_async_copy(rows[p], acc_sh.at[dstv[p]], ss[p]).wait()

        pltpu.sync_copy(zfeat, rw0)

        def zblk(k, carry):
            pltpu.sync_copy(rw0, acc_sh.at[pl.ds(row0 + k * CHUNK, CHUNK)])
            return carry

        lax.fori_loop(0, nsub, zblk, 0)
        plsc.subcore_barrier()

        issue_src(0, 0)
        issue_dst(0, 0)
        issue_src(1, 1)
        wait_src(0)
        issue_gather(0)
        wait_src(1)
        issue_dst(1, 1)
        issue_gather(1)
        wait_gather(0)
        issue_src(2, 0)
        wait_dst(0)
        issue_scatter(0)

        def pair(mm, carry):
            for t in range(2):
                p, q = 1 - t, t
                m = 2 * mm + 1 + t
                wait_scatter(q)
                issue_dst(m + 1, q)
                wait_src(q)
                issue_gather(q)
                wait_gather(p)
                issue_src(m + 2, p)
                wait_dst(p)
                issue_scatter(p)
            return carry

        lax.fori_loop(0, (N_CHUNKS - 2) // 2, pair, 0)
        wait_scatter(0)
        wait_gather(1)
        wait_dst(1)
        issue_scatter(1)
        wait_scatter(1)
        wait_src(0)
        plsc.subcore_barrier()
        out0 = c * ACC_ROWS + row0

        def oblk(k, carry):
            pltpu.sync_copy(acc_sh.at[pl.ds(row0 + k * CHUNK, CHUNK)], rw0)
            pltpu.sync_copy(rw0, sums_out.at[pl.ds(out0 + k * CHUNK, CHUNK)])
            return carry

        lax.fori_loop(0, nsub, oblk, 0)

    return pl.kernel(body, mesh=_MESH, out_type=out_type, scratch_types=scratch)


def _sc_cnt():
    out_type = [jax.ShapeDtypeStruct((2 * ACC_ROWS, FEAT), jnp.float32)]
    scratch = [
        pltpu.VMEM_SHARED((ACC_ROWS, FEAT), jnp.float32),
        pltpu.VMEM((CHUNK,), jnp.int32),
        pltpu.VMEM((CHUNK,), jnp.int32),
        pltpu.VMEM((CHUNK, FEAT), jnp.float32),
    ] + [pltpu.SemaphoreType.DMA] * 4

    def body(dst_hbm, zfeat, ones_hbm, cnts_out, acc_sh, dv0, dv1, rows_v,
             qd0, qd1, qc0, qc1):
        dstv, si_d, ss = (dv0, dv1), (qd0, qd1), (qc0, qc1)
        c = lax.axis_index("c")
        s = lax.axis_index("s")
        tid = c * 16 + s
        row0 = s * ROWS_PER_TILE
        nsub = ROWS_PER_TILE // CHUNK

        def issue_dst(m, p):
            pltpu.async_copy(dst_hbm.at[tid * N_CHUNKS + m], dstv[p], si_d[p])

        def wait_dst(p):
            pltpu.make_async_copy(dst_hbm.at[0], dstv[p], si_d[p]).wait()

        def issue_scatter(p):
            pltpu.async_copy(rows_v, acc_sh.at[dstv[p]], ss[p], add=True)

        def wait_scatter(p):
            pltpu.make_async_copy(rows_v, acc_sh.at[dstv[p]], ss[p]).wait()

        pltpu.sync_copy(zfeat, rows_v)

        def zblk(k, carry):
            pltpu.sync_copy(rows_v, acc_sh.at[pl.ds(row0 + k * CHUNK, CHUNK)])
            return carry

        lax.fori_loop(0, nsub, zblk, 0)
        pltpu.sync_copy(ones_hbm, rows_v)
        plsc.subcore_barrier()

        issue_dst(0, 0)
        wait_dst(0)
        issue_scatter(0)
        issue_dst(1, 1)

        def pair(mm, carry):
            for t in range(2):
                p, q = 1 - t, t
                wait_dst(p)
                issue_scatter(p)
                wait_scatter(q)
                issue_dst(2 * mm + 2 + t, q)
            return carry

        lax.fori_loop(0, (N_CHUNKS - 2) // 2, pair, 0)
        wait_dst(1)
        issue_scatter(1)
        wait_scatter(0)
        wait_scatter(1)
        plsc.subcore_barrier()
        out0 = c * ACC_ROWS + row0

        def oblk(k, carry):
            pltpu.sync_copy(acc_sh.at[pl.ds(row0 + k * CHUNK, CHUNK)], rows_v)
            pltpu.sync_copy(rows_v, cnts_out.at[pl.ds(out0 + k * CHUNK, CHUNK)])
            return carry

        lax.fori_loop(0, nsub, oblk, 0)

    return pl.kernel(body, mesh=_MESH, out_type=out_type, scratch_types=scratch)


def _tc1_body(sums_ref, cnts_ref, w_ref, out_ref):
    s = sums_ref[0][:N_NODES]
    cnt = cnts_ref[0][:N_NODES, 0:1]
    m = s / jnp.maximum(cnt, 1.0)
    out_ref[0] = jnp.maximum(
        jnp.dot(m, w_ref[0], preferred_element_type=jnp.float32), 0.0)


def _tc2_body(sums_ref, cnts_ref, w1_ref, wfc_ref, b_ref, ft_ref, out_ref):
    s = sums_ref[0][:N_NODES]
    cnt = cnts_ref[0][:N_NODES, 0:1]
    m = s / jnp.maximum(cnt, 1.0)
    v = jnp.maximum(jnp.dot(m, w1_ref[0], preferred_element_type=jnp.float32), 0.0)
    o = (jnp.dot(v, wfc_ref[0][:FEAT], preferred_element_type=jnp.float32)
         + jnp.dot(ft_ref[0], wfc_ref[0][FEAT:], preferred_element_type=jnp.float32)
         + b_ref[0, 0])
    out_ref[0] = o


def _pad_edges(idx, pad_val):
    pad = jnp.full((E_PAD - idx.shape[0],), pad_val, jnp.int32)
    return jnp.concatenate([idx.astype(jnp.int32), pad]).reshape(
        16 * N_CHUNKS, CHUNK)


def kernel(ft_a, ft_p, edge_a2p, edge_p2a, W0_ap, W0_pa, W1_ap, W1_pa,
           Wfc_a, bfc_a, Wfc_p, bfc_p):
    f32 = jnp.float32
    src_all = jnp.concatenate([
        _pad_edges(edge_a2p[1], 0),
        _pad_edges(edge_p2a[1] + N_NODES, N_NODES),
        jnp.zeros((16, CHUNK), jnp.int32),
    ])
    dst_all = jnp.concatenate([
        _pad_edges(edge_a2p[0], PAD_DST),
        _pad_edges(edge_p2a[0], PAD_DST),
    ])

    zfeat = jnp.zeros((CHUNK, FEAT), f32)
    ones = jnp.ones((CHUNK, FEAT), f32)

    table1 = jnp.concatenate([ft_p, ft_a], axis=0)
    sums1 = _sc_agg()(table1, src_all, dst_all, zfeat)[0]
    cnts = _sc_cnt()(dst_all, zfeat, ones)[0]

    sums1_r = sums1.reshape(2, ACC_ROWS, FEAT)
    cnts_r = cnts.reshape(2, ACC_ROWS, FEAT)
    w0 = jnp.stack([W0_pa, W0_ap]).reshape(2, FEAT, FEAT)
    flip = lambda i: (1 - i, 0, 0)
    ident = lambda i: (i, 0, 0)
    table2 = pl.pallas_call(
        _tc1_body,
        grid=(2,),
        in_specs=[
            pl.BlockSpec((1, ACC_ROWS, FEAT), flip),
            pl.BlockSpec((1, ACC_ROWS, FEAT), flip),
            pl.BlockSpec((1, FEAT, FEAT), ident),
        ],
        out_specs=pl.BlockSpec((1, N_NODES, FEAT), ident),
        out_shape=jax.ShapeDtypeStruct((2, N_NODES, FEAT), f32),
    )(sums1_r, cnts_r, w0)

    sums2 = _sc_agg()(table2.reshape(2 * N_NODES, FEAT), src_all, dst_all,
                      zfeat)[0]

    sums2_r = sums2.reshape(2, ACC_ROWS, FEAT)
    w1 = jnp.stack([W1_ap, W1_pa])
    wfc = jnp.stack([Wfc_a, Wfc_p])
    bfc = jnp.stack([bfc_a, bfc_p]).reshape(2, 1, FEAT)
    ft = jnp.stack([ft_a, ft_p])
    out = pl.pallas_call(
        _tc2_body,
        grid=(2,),
        in_specs=[
            pl.BlockSpec((1, ACC_ROWS, FEAT), ident),
            pl.BlockSpec((1, ACC_ROWS, FEAT), ident),
            pl.BlockSpec((1, FEAT, FEAT), ident),
            pl.BlockSpec((1, 2 * FEAT, FEAT), ident),
            pl.BlockSpec((1, 1, FEAT), ident),
            pl.BlockSpec((1, N_NODES, FEAT), ident),
        ],
        out_specs=pl.BlockSpec((1, N_NODES, FEAT), ident),
        out_shape=jax.ShapeDtypeStruct((2, N_NODES, FEAT), f32),
    )(sums2_r, cnts_r, w1, wfc, bfc, ft)
    return out.reshape(2 * N_NODES, FEAT)
